# baseline (speedup 1.0000x reference)
import math

import numpy as np
import jax
import jax.numpy as jnp
from jax import lax
from jax.experimental import pallas as pl
from jax.experimental.pallas import tpu as pltpu
from jax.experimental.pallas import tpu_sc as plsc
from jax.scipy.special import gammaln

_C = 20
_B = 1048576
_NC = 2
_NS = 16
_NW = _NC * _NS
_ROWS_W = _B // _NW
_R_C = 512
_CHUNKS = _ROWS_W // _R_C
_GROUPS = _R_C // 16

_LN2 = math.log(2.0)


def _soft_weight_table():
    n = jnp.float32(_C - 1)
    ks = jnp.arange(_C, dtype=jnp.float32)
    ps = ks / n
    eps = jnp.float32(1e-5)
    zero = jnp.float32(0.0)
    mu = ks
    alpha = jnp.sqrt(jnp.maximum(mu * (1.0 - ps) - 1.0, zero)
                     / (jnp.maximum(mu, eps) * (1.0 + mu / jnp.maximum(n - mu, eps))))
    mu_p = mu[:, None, None]
    ks_p = ks[None, :, None]
    i_p = ks[None, None, :]
    ps2 = jnp.stack([ps + alpha, ps - mu * alpha / jnp.maximum(n - mu, eps)], axis=0)
    valid = jnp.logical_and(i_p <= mu_p, i_p >= mu_p + ks_p - n)
    validf = valid.astype(jnp.float32)
    binomials = jnp.exp(
        gammaln(n - mu_p + 1.0) + gammaln(mu_p + 1.0)
        - gammaln(jnp.maximum(ks_p - i_p + 1.0, 1.0))
        - gammaln(i_p + 1.0)
        - gammaln(jnp.maximum(mu_p - i_p + 1.0, 1.0))
        - gammaln(jnp.maximum(n - mu_p - ks_p + i_p + 1.0, 1.0))
    ) * validf
    p = ps2[:, :, None, None]
    stable = jnp.logical_not(jnp.logical_or(jnp.isclose(p, 0.0), jnp.isclose(p, 1.0)))
    sn = stable.astype(jnp.float32)
    p = jnp.where(stable, p, 0.5)
    products = jnp.exp(
        (jnp.log(p[0]) * i_p
         + jnp.log(1.0 - p[0]) * (mu_p - i_p)
         + jnp.log(p[1]) * (ks_p - i_p) * sn[0]
         + jnp.log(1.0 - p[1]) * (n - mu_p - ks_p + i_p))
        * sn[1] * validf
    )
    return (binomials * products).sum(axis=-1)


_FJ = (np.arange(8192, dtype=np.float64) + 0.5) / 8192.0
_CTAB = (np.log2(1.0 + _FJ) - _FJ - 127.0).astype(np.float32)


def _sc_body(x_hbm, t_hbm, sw_hbm, ct_hbm, out_hbm,
             x_v, t_v, sw_v, ct_v, res_v):
    cid = lax.axis_index("c")
    sid = lax.axis_index("s")
    wid = sid * _NC + cid
    base = wid * _ROWS_W

    pltpu.sync_copy(sw_hbm, sw_v)
    pltpu.sync_copy(ct_hbm, ct_v)

    lane = lax.iota(jnp.int32, 16)
    lane20 = lane * _C
    scale = jnp.float32(2.0 ** -23)
    epsv = jnp.float32(1e-16)

    def chunk_body(ci, acc):
        rbase = base + ci * _R_C
        pltpu.sync_copy(x_hbm.at[pl.ds(rbase, _R_C), :], x_v)
        pltpu.sync_copy(t_hbm.at[pl.ds(rbase, _R_C)], t_v)

        def grp_body(g, accv):
            t16 = t_v[pl.ds(g * 16, 16)]
            t20 = t16 * _C
            rows = lane + g * 16
            for c in range(_C):
                cvec = jnp.full((16,), c, jnp.int32)
                xv = plsc.load_gather(x_v, [rows, cvec])
                wv = plsc.load_gather(sw_v, [t20 + c])
                xe = xv + epsv
                i = lax.bitcast_convert_type(xe, jnp.int32)
                fi = lax.convert_element_type(i, jnp.float32)
                jdx = (i >> 10) & 0x1FFF
                corr = plsc.load_gather(ct_v, [jdx])
                lg2 = fi * scale + corr
                accv = accv + lg2 * wv
            return accv

        return lax.fori_loop(0, _GROUPS, grp_body, acc)

    acc = lax.fori_loop(0, _CHUNKS, chunk_body, jnp.zeros((16,), jnp.float32))
    res_v[...] = acc
    pltpu.sync_copy(res_v, out_hbm.at[wid])


_TC_BLK = 4096


_G4 = (-0.0791500542040274, 0.31221242563682267, -0.6695129688738825,
       0.4360971979913858, 0.0002042532880674506 - 127.0)


def _tc_body(x_ref, t_ref, sw_ref, out_ref):
    i = pl.program_id(0)
    x = x_ref[...]
    xe = x + jnp.float32(1e-16)
    bits = lax.bitcast_convert_type(xe, jnp.int32)
    fi = lax.convert_element_type(bits, jnp.float32)
    fm = lax.convert_element_type(bits & 0x7FFFFF, jnp.float32) * jnp.float32(2.0 ** -23)
    p = jnp.float32(_G4[0])
    for cc in _G4[1:]:
        p = p * fm + jnp.float32(cc)
    lg = fi * jnp.float32(2.0 ** -23) + p
    tt = t_ref[0, 0, :]
    iota2 = lax.broadcasted_iota(jnp.int32, (_TC_BLK, _C), 1)
    oh = (tt[:, None] == iota2).astype(jnp.float32)
    w = jnp.dot(oh, sw_ref[...], preferred_element_type=jnp.float32)
    part = jnp.sum(lg * w)

    @pl.when(i == 0)
    def _():
        out_ref[0, 0] = jnp.float32(0.0)

    out_ref[0, 0] += part


def _tc_kernel(inputs, targets, sw):
    nb = _B // _TC_BLK
    t3 = targets.reshape(nb, 1, _TC_BLK)
    total = pl.pallas_call(
        _tc_body,
        grid=(nb,),
        in_specs=[
            pl.BlockSpec((_TC_BLK, _C), lambda i: (i, 0)),
            pl.BlockSpec((1, 1, _TC_BLK), lambda i: (i, 0, 0)),
            pl.BlockSpec((_C, _C), lambda i: (0, 0)),
        ],
        out_specs=pl.BlockSpec((1, 1), lambda i: (0, 0),
                               memory_space=pltpu.SMEM),
        out_shape=jax.ShapeDtypeStruct((1, 1), jnp.float32),
    )(inputs, t3, sw)
    return total[0, 0]


def kernel(inputs, targets):
    sw = _soft_weight_table() * jnp.float32(_LN2)
    total = _tc_kernel(inputs, targets, sw)
    return -(total / _B) - jnp.float32(1.0)


def _sc_kernel_unused(inputs, targets):
    sw = (_soft_weight_table() * jnp.float32(_LN2)).reshape(_C * _C)
    ct = jnp.asarray(_CTAB)

    mesh = plsc.VectorSubcoreMesh(core_axis_name="c", subcore_axis_name="s",
                                  num_cores=_NC, num_subcores=_NS)
    parts = pl.kernel(
        _sc_body,
        out_type=jax.ShapeDtypeStruct((_NW, 16), jnp.float32),
        mesh=mesh,
        compiler_params=pltpu.CompilerParams(needs_layout_passes=False),
        scratch_types=[
            pltpu.VMEM((_R_C, _C), jnp.float32),
            pltpu.VMEM((_R_C,), jnp.int32),
            pltpu.VMEM((_C * _C,), jnp.float32),
            pltpu.VMEM((8192,), jnp.float32),
            pltpu.VMEM((16,), jnp.float32),
        ],
    )(inputs, targets, sw, ct)
    total = jnp.sum(parts)
    return -(total / _B) - jnp.float32(1.0)


# speedup vs baseline: 1.1328x; 1.1328x over previous
import math

import numpy as np
import jax
import jax.numpy as jnp
from jax import lax
from jax.experimental import pallas as pl
from jax.experimental.pallas import tpu as pltpu
from jax.experimental.pallas import tpu_sc as plsc
from jax.scipy.special import gammaln

_C = 20
_B = 1048576
_B_SC = 524288
_NC = 2
_NS = 16
_NW = _NC * _NS
_ROWS_W = _B_SC // _NW
_R_C = 512
_CHUNKS = _ROWS_W // _R_C
_GROUPS = _R_C // 16

_LN2 = math.log(2.0)


def _soft_weight_table():
    n = jnp.float32(_C - 1)
    ks = jnp.arange(_C, dtype=jnp.float32)
    ps = ks / n
    eps = jnp.float32(1e-5)
    zero = jnp.float32(0.0)
    mu = ks
    alpha = jnp.sqrt(jnp.maximum(mu * (1.0 - ps) - 1.0, zero)
                     / (jnp.maximum(mu, eps) * (1.0 + mu / jnp.maximum(n - mu, eps))))
    mu_p = mu[:, None, None]
    ks_p = ks[None, :, None]
    i_p = ks[None, None, :]
    ps2 = jnp.stack([ps + alpha, ps - mu * alpha / jnp.maximum(n - mu, eps)], axis=0)
    valid = jnp.logical_and(i_p <= mu_p, i_p >= mu_p + ks_p - n)
    validf = valid.astype(jnp.float32)
    binomials = jnp.exp(
        gammaln(n - mu_p + 1.0) + gammaln(mu_p + 1.0)
        - gammaln(jnp.maximum(ks_p - i_p + 1.0, 1.0))
        - gammaln(i_p + 1.0)
        - gammaln(jnp.maximum(mu_p - i_p + 1.0, 1.0))
        - gammaln(jnp.maximum(n - mu_p - ks_p + i_p + 1.0, 1.0))
    ) * validf
    p = ps2[:, :, None, None]
    stable = jnp.logical_not(jnp.logical_or(jnp.isclose(p, 0.0), jnp.isclose(p, 1.0)))
    sn = stable.astype(jnp.float32)
    p = jnp.where(stable, p, 0.5)
    products = jnp.exp(
        (jnp.log(p[0]) * i_p
         + jnp.log(1.0 - p[0]) * (mu_p - i_p)
         + jnp.log(p[1]) * (ks_p - i_p) * sn[0]
         + jnp.log(1.0 - p[1]) * (n - mu_p - ks_p + i_p))
        * sn[1] * validf
    )
    return (binomials * products).sum(axis=-1)


_FJ = (np.arange(8192, dtype=np.float64) + 0.5) / 8192.0
_CTAB = (np.log2(1.0 + _FJ) - _FJ - 127.0).astype(np.float32)


def _sc_body(x_hbm, t_hbm, sw_hbm, ct_hbm, out_hbm,
             x_v, t_v, sw_v, ct_v, res_v):
    cid = lax.axis_index("c")
    sid = lax.axis_index("s")
    wid = sid * _NC + cid
    base = wid * _ROWS_W

    pltpu.sync_copy(sw_hbm, sw_v)
    pltpu.sync_copy(ct_hbm, ct_v)

    lane = lax.iota(jnp.int32, 16)
    lane20 = lane * _C
    scale = jnp.float32(2.0 ** -23)
    epsv = jnp.float32(1e-16)

    def chunk_body(ci, acc):
        rbase = base + ci * _R_C
        pltpu.sync_copy(x_hbm.at[pl.ds(rbase, _R_C), :], x_v)
        pltpu.sync_copy(t_hbm.at[pl.ds(rbase, _R_C)], t_v)

        def grp_body(g, accv):
            t16 = t_v[pl.ds(g * 16, 16)]
            t20 = t16 * _C
            rows = lane + g * 16
            for c in range(_C):
                cvec = jnp.full((16,), c, jnp.int32)
                xv = plsc.load_gather(x_v, [rows, cvec])
                wv = plsc.load_gather(sw_v, [t20 + c])
                xe = xv + epsv
                i = lax.bitcast_convert_type(xe, jnp.int32)
                fi = lax.convert_element_type(i, jnp.float32)
                jdx = (i >> 10) & 0x1FFF
                corr = plsc.load_gather(ct_v, [jdx])
                lg2 = fi * scale + corr
                accv = accv + lg2 * wv
            return accv

        return lax.fori_loop(0, _GROUPS, grp_body, acc)

    acc = lax.fori_loop(0, _CHUNKS, chunk_body, jnp.zeros((16,), jnp.float32))
    res_v[...] = acc
    pltpu.sync_copy(res_v, out_hbm.at[wid])


_TC_BLK = 4096


_G4 = (-0.0791500542040274, 0.31221242563682267, -0.6695129688738825,
       0.4360971979913858, 0.0002042532880674506 - 127.0)


def _tc_body(x_ref, t_ref, sw_ref, out_ref):
    i = pl.program_id(0)
    x = x_ref[...]
    lg = jnp.log(x + jnp.float32(1e-16))
    tt = t_ref[0, 0, :]
    iota2 = lax.broadcasted_iota(jnp.int32, (_TC_BLK, _C), 1)
    oh = (tt[:, None] == iota2).astype(jnp.float32)
    w = jnp.dot(oh, sw_ref[...], preferred_element_type=jnp.float32)
    part = jnp.sum(lg * w)

    @pl.when(i == 0)
    def _():
        out_ref[0, 0] = jnp.float32(0.0)

    out_ref[0, 0] += part


def _tc_kernel(inputs, targets, sw):
    nb = (_B - _B_SC) // _TC_BLK
    off = _B_SC // _TC_BLK
    t3 = targets.reshape(_B // _TC_BLK, 1, _TC_BLK)
    total = pl.pallas_call(
        _tc_body,
        grid=(nb,),
        in_specs=[
            pl.BlockSpec((_TC_BLK, _C), lambda i: (i + off, 0)),
            pl.BlockSpec((1, 1, _TC_BLK), lambda i: (i + off, 0, 0)),
            pl.BlockSpec((_C, _C), lambda i: (0, 0)),
        ],
        out_specs=pl.BlockSpec((1, 1), lambda i: (0, 0),
                               memory_space=pltpu.SMEM),
        out_shape=jax.ShapeDtypeStruct((1, 1), jnp.float32),
    )(inputs, t3, sw)
    return total[0, 0]


def _sc_kernel(inputs, targets, sw_ln2_flat, ct):
    mesh = plsc.VectorSubcoreMesh(core_axis_name="c", subcore_axis_name="s",
                                  num_cores=_NC, num_subcores=_NS)
    parts = pl.kernel(
        _sc_body,
        out_type=jax.ShapeDtypeStruct((_NW, 16), jnp.float32),
        mesh=mesh,
        compiler_params=pltpu.CompilerParams(needs_layout_passes=False),
        scratch_types=[
            pltpu.VMEM((_R_C, _C), jnp.float32),
            pltpu.VMEM((_R_C,), jnp.int32),
            pltpu.VMEM((_C * _C,), jnp.float32),
            pltpu.VMEM((8192,), jnp.float32),
            pltpu.VMEM((16,), jnp.float32),
        ],
    )(inputs, targets, sw_ln2_flat, ct)
    return jnp.sum(parts)


def kernel(inputs, targets):
    sw = _soft_weight_table()
    sw_ln2 = sw * jnp.float32(_LN2)
    ct = jnp.asarray(_CTAB)
    total_sc = _sc_kernel(inputs, targets, sw_ln2.reshape(_C * _C), ct)
    total_tc = _tc_kernel(inputs, targets, sw)
    return -((total_sc + total_tc) / _B) - jnp.float32(1.0)


# speedup vs baseline: 1.2985x; 1.1463x over previous
import math

import numpy as np
import jax
import jax.numpy as jnp
from jax import lax
from jax.experimental import pallas as pl
from jax.experimental.pallas import tpu as pltpu
from jax.experimental.pallas import tpu_sc as plsc
from jax.scipy.special import gammaln

_C = 20
_B = 1048576
_B_SC = 360448
_NC = 2
_NS = 16
_NW = _NC * _NS
_ROWS_W = _B_SC // _NW
_R_C = 512
_CHUNKS = _ROWS_W // _R_C
_GROUPS = _R_C // 16

_LN2 = math.log(2.0)


def _soft_weight_table():
    n = jnp.float32(_C - 1)
    ks = jnp.arange(_C, dtype=jnp.float32)
    ps = ks / n
    eps = jnp.float32(1e-5)
    zero = jnp.float32(0.0)
    mu = ks
    alpha = jnp.sqrt(jnp.maximum(mu * (1.0 - ps) - 1.0, zero)
                     / (jnp.maximum(mu, eps) * (1.0 + mu / jnp.maximum(n - mu, eps))))
    mu_p = mu[:, None, None]
    ks_p = ks[None, :, None]
    i_p = ks[None, None, :]
    ps2 = jnp.stack([ps + alpha, ps - mu * alpha / jnp.maximum(n - mu, eps)], axis=0)
    valid = jnp.logical_and(i_p <= mu_p, i_p >= mu_p + ks_p - n)
    validf = valid.astype(jnp.float32)
    binomials = jnp.exp(
        gammaln(n - mu_p + 1.0) + gammaln(mu_p + 1.0)
        - gammaln(jnp.maximum(ks_p - i_p + 1.0, 1.0))
        - gammaln(i_p + 1.0)
        - gammaln(jnp.maximum(mu_p - i_p + 1.0, 1.0))
        - gammaln(jnp.maximum(n - mu_p - ks_p + i_p + 1.0, 1.0))
    ) * validf
    p = ps2[:, :, None, None]
    stable = jnp.logical_not(jnp.logical_or(jnp.isclose(p, 0.0), jnp.isclose(p, 1.0)))
    sn = stable.astype(jnp.float32)
    p = jnp.where(stable, p, 0.5)
    products = jnp.exp(
        (jnp.log(p[0]) * i_p
         + jnp.log(1.0 - p[0]) * (mu_p - i_p)
         + jnp.log(p[1]) * (ks_p - i_p) * sn[0]
         + jnp.log(1.0 - p[1]) * (n - mu_p - ks_p + i_p))
        * sn[1] * validf
    )
    return (binomials * products).sum(axis=-1)


_FJ = (np.arange(8192, dtype=np.float64) + 0.5) / 8192.0
_CTAB = (np.log2(1.0 + _FJ) - _FJ - 127.0).astype(np.float32)


def _sc_body(x_hbm, t_hbm, sw_hbm, ct_hbm, out_hbm,
             x_v, t_v, sw_v, ct_v, res_v):
    cid = lax.axis_index("c")
    sid = lax.axis_index("s")
    wid = sid * _NC + cid
    base = wid * _ROWS_W

    pltpu.sync_copy(sw_hbm, sw_v)
    pltpu.sync_copy(ct_hbm, ct_v)

    lane = lax.iota(jnp.int32, 16)
    lane20 = lane * _C
    scale = jnp.float32(2.0 ** -23)
    epsv = jnp.float32(1e-16)

    def chunk_body(ci, acc):
        rbase = base + ci * _R_C
        pltpu.sync_copy(x_hbm.at[pl.ds(rbase, _R_C), :], x_v)
        pltpu.sync_copy(t_hbm.at[pl.ds(rbase, _R_C)], t_v)

        def grp_body(g, accv):
            t16 = t_v[pl.ds(g * 16, 16)]
            t20 = t16 * _C
            rows = lane + g * 16
            for c in range(_C):
                cvec = jnp.full((16,), c, jnp.int32)
                xv = plsc.load_gather(x_v, [rows, cvec])
                wv = plsc.load_gather(sw_v, [t20 + c])
                xe = xv + epsv
                i = lax.bitcast_convert_type(xe, jnp.int32)
                fi = lax.convert_element_type(i, jnp.float32)
                jdx = (i >> 10) & 0x1FFF
                corr = plsc.load_gather(ct_v, [jdx])
                lg2 = fi * scale + corr
                accv = accv + lg2 * wv
            return accv

        return lax.fori_loop(0, _GROUPS, grp_body, acc)

    acc = lax.fori_loop(0, _CHUNKS, chunk_body, jnp.zeros((16,), jnp.float32))
    res_v[...] = acc
    pltpu.sync_copy(res_v, out_hbm.at[wid])


_TC_BLK = 4096


_G4 = (-0.0791500542040274, 0.31221242563682267, -0.6695129688738825,
       0.4360971979913858, 0.0002042532880674506 - 127.0)


def _tc_body(x_ref, t_ref, sw_ref, out_ref):
    i = pl.program_id(0)
    x = x_ref[...]
    lg = jnp.log(x + jnp.float32(1e-16))
    tt = t_ref[0, 0, :]
    iota2 = lax.broadcasted_iota(jnp.int32, (_TC_BLK, _C), 1)
    oh = (tt[:, None] == iota2).astype(jnp.float32)
    w = jnp.dot(oh, sw_ref[...], preferred_element_type=jnp.float32)
    part = jnp.sum(lg * w)

    @pl.when(i == 0)
    def _():
        out_ref[0, 0] = jnp.float32(0.0)

    out_ref[0, 0] += part


def _tc_kernel(inputs, targets, sw):
    nb = (_B - _B_SC) // _TC_BLK
    off = _B_SC // _TC_BLK
    t3 = targets.reshape(_B // _TC_BLK, 1, _TC_BLK)
    total = pl.pallas_call(
        _tc_body,
        grid=(nb,),
        in_specs=[
            pl.BlockSpec((_TC_BLK, _C), lambda i: (i + off, 0)),
            pl.BlockSpec((1, 1, _TC_BLK), lambda i: (i + off, 0, 0)),
            pl.BlockSpec((_C, _C), lambda i: (0, 0)),
        ],
        out_specs=pl.BlockSpec((1, 1), lambda i: (0, 0),
                               memory_space=pltpu.SMEM),
        out_shape=jax.ShapeDtypeStruct((1, 1), jnp.float32),
    )(inputs, t3, sw)
    return total[0, 0]


def _sc_kernel(inputs, targets, sw_ln2_flat, ct):
    mesh = plsc.VectorSubcoreMesh(core_axis_name="c", subcore_axis_name="s",
                                  num_cores=_NC, num_subcores=_NS)
    parts = pl.kernel(
        _sc_body,
        out_type=jax.ShapeDtypeStruct((_NW, 16), jnp.float32),
        mesh=mesh,
        compiler_params=pltpu.CompilerParams(needs_layout_passes=False),
        scratch_types=[
            pltpu.VMEM((_R_C, _C), jnp.float32),
            pltpu.VMEM((_R_C,), jnp.int32),
            pltpu.VMEM((_C * _C,), jnp.float32),
            pltpu.VMEM((8192,), jnp.float32),
            pltpu.VMEM((16,), jnp.float32),
        ],
    )(inputs, targets, sw_ln2_flat, ct)
    return jnp.sum(parts)


def kernel(inputs, targets):
    sw = _soft_weight_table()
    sw_ln2 = sw * jnp.float32(_LN2)
    ct = jnp.asarray(_CTAB)
    total_sc = _sc_kernel(inputs, targets, sw_ln2.reshape(_C * _C), ct)
    total_tc = _tc_kernel(inputs, targets, sw)
    return -((total_sc + total_tc) / _B) - jnp.float32(1.0)
